# Initial kernel scaffold; baseline (speedup 1.0000x reference)
import jax
import jax.numpy as jnp
from jax.experimental import pallas as pl


def kernel(x, prev_output, weight_matrix):
    raise NotImplementedError("write your pallas kernel here")


# speedup vs baseline: 8.5348x; 8.5348x over previous
import functools

import jax
import jax.numpy as jnp
from jax import lax
from jax.experimental import pallas as pl
from jax.experimental.pallas import tpu as pltpu
from jax.experimental.pallas import tpu_sc as plsc

_L = 16


def _body(x_hbm, prev_hbm, w_hbm, out_hbm,
          prev_v, x_v, w_v, s_v, o_v, idx_v, hist1_v, hist2_v, sem,
          *, rows_per_w, nv, cfull, crem, k_drop):
    lanes = jnp.arange(_L, dtype=jnp.int32)
    ones = jnp.ones((_L,), jnp.int32)
    ninf = jnp.full((_L,), -jnp.inf, jnp.float32)
    C = cfull * _L + crem
    BIGI = jnp.int32(1 << 20)

    wid = lax.axis_index("s") * 2 + lax.axis_index("c")
    base = wid * rows_per_w

    def scan_top(hist_ref, nvregs, k_need):
        def cond(c):
            j, acc, found, p, kr = c
            return jnp.logical_not(found) & (j >= 0)

        def body(c):
            j, acc, found, p, kr = c
            h = hist_ref[pl.ds(j * _L, _L)]
            pre = plsc.cumsum(h)
            tot = jnp.max(pre)
            S = (acc + tot) - pre + h
            found_now = (acc + tot) >= kr
            mask = S >= kr
            cnt = jnp.max(plsc.all_reduce_population_count(mask))
            i = cnt - 1
            S_at = jnp.min(jnp.where(mask, S, jnp.int32(1 << 30)))
            h_at = jnp.max(jnp.where(lanes == i, h, 0))
            p_new = j * _L + i
            kr_new = kr - (S_at - h_at)
            return (j - 1, acc + tot, found_now,
                    jnp.where(found_now, p_new, p),
                    jnp.where(found_now, kr_new, kr))

        init = (jnp.int32(nvregs - 1), jnp.int32(0), jnp.bool_(False),
                jnp.int32(0), k_need)
        _, _, _, p, kr = lax.while_loop(cond, body, init)
        return p, kr

    def clr1(j, _):
        hist1_v[pl.ds(j * _L, _L)] = jnp.zeros((_L,), jnp.int32)
        return 0

    def clr2(j, _):
        hist2_v[pl.ds(j * _L, _L)] = jnp.zeros((_L,), jnp.int32)
        return 0

    def row_step(i, _):
        r = base + i
        pltpu.sync_copy(prev_hbm.at[r], prev_v)

        def t2(j, c):
            a1, a2 = c
            v = prev_v[pl.ds(j * _L, _L)]
            a2 = jnp.maximum(a2, jnp.minimum(a1, v))
            a1 = jnp.maximum(a1, v)
            return a1, a2

        a1, a2 = lax.fori_loop(0, cfull, t2, (ninf, ninf))
        if crem:
            vt = prev_v[pl.ds(C - _L, _L)]
            vt = jnp.where(lanes >= (_L - crem), vt, ninf)
            a2 = jnp.maximum(a2, jnp.minimum(a1, vt))
            a1 = jnp.maximum(a1, vt)
        m1 = jnp.max(a1)
        f1 = jnp.max(plsc.all_reduce_ffs(a1 == m1))
        a1x = jnp.where(lanes == f1, ninf, a1)
        m2 = jnp.maximum(jnp.max(a1x), jnp.max(a2))

        nsweep = cfull + (1 if crem else 0)

        def idx_sweep(match_val, excl):
            def bodyf(j, c):
                off = jnp.where(j < cfull, j * _L, C - _L)
                lo = jnp.where(j < cfull, 0, _L - crem)
                v = prev_v[pl.ds(off, _L)]
                iv = lanes + off
                ok = (v == match_val) & (lanes >= lo) & (iv != excl)
                cand = jnp.where(ok, iv, BIGI)
                return jnp.minimum(c, jnp.min(cand))
            return lax.fori_loop(0, nsweep, bodyf, BIGI)

        i1 = idx_sweep(m1, jnp.int32(-1))
        i2 = idx_sweep(m2, i1)
        iv2 = jnp.where(lanes == 0, i1, i2)
        plsc.store_scatter(idx_v, [lanes], iv2, mask=lanes < 2)
        gather = pltpu.async_copy(w_hbm.at[idx_v], w_v, sem)
        pltpu.sync_copy(x_hbm.at[r], x_v)
        gather.wait()

        lax.fori_loop(0, hist1_v.shape[0] // _L, clr1, 0)
        lax.fori_loop(0, hist2_v.shape[0] // _L, clr2, 0)

        def sc_pass(j, _):
            off = j * _L
            xv = x_v[pl.ds(off, _L)]
            dv = w_v[0, pl.ds(off, _L)] - w_v[1, pl.ds(off, _L)]
            s = jnp.abs(xv * dv)
            s_v[pl.ds(off, _L)] = s
            bits = plsc.bitcast(s, jnp.uint32)
            bk = (bits >> 20).astype(jnp.int32)
            plsc.addupdate_scatter(hist1_v, [bk], ones)
            return 0

        lax.fori_loop(0, nv, sc_pass, 0)
        p1, k1 = scan_top(hist1_v, hist1_v.shape[0] // _L, jnp.int32(k_drop))
        p1u = p1.astype(jnp.uint32)

        def l2_pass(j, _):
            s = s_v[pl.ds(j * _L, _L)]
            bits = plsc.bitcast(s, jnp.uint32)
            m = (bits >> 20) == p1u
            bk = ((bits >> 10) & jnp.uint32(1023)).astype(jnp.int32)
            plsc.addupdate_scatter(hist2_v, [bk], ones, mask=m)
            return 0

        lax.fori_loop(0, nv, l2_pass, 0)
        p2, k2 = scan_top(hist2_v, hist2_v.shape[0] // _L, k1)
        pfx2 = (p1u << jnp.uint32(10)) | p2.astype(jnp.uint32)

        lax.fori_loop(0, hist2_v.shape[0] // _L, clr2, 0)

        def l3_pass(j, _):
            s = s_v[pl.ds(j * _L, _L)]
            bits = plsc.bitcast(s, jnp.uint32)
            m = (bits >> 10) == pfx2
            bk = (bits & jnp.uint32(1023)).astype(jnp.int32)
            plsc.addupdate_scatter(hist2_v, [bk], ones, mask=m)
            return 0

        lax.fori_loop(0, nv, l3_pass, 0)
        p3, _ = scan_top(hist2_v, hist2_v.shape[0] // _L, k2)
        thr = (pfx2 << jnp.uint32(10)) | p3.astype(jnp.uint32)

        def ap(j, _):
            off = j * _L
            s = s_v[pl.ds(off, _L)]
            bits = plsc.bitcast(s, jnp.uint32)
            xv = x_v[pl.ds(off, _L)]
            o_v[pl.ds(off, _L)] = jnp.where(bits < thr, xv, jnp.float32(0))
            return 0

        lax.fori_loop(0, nv, ap, 0)
        pltpu.sync_copy(o_v, out_hbm.at[r])
        return 0

    lax.fori_loop(0, rows_per_w, row_step, 0)


@jax.jit
def kernel(x, prev_output, weight_matrix):
    B, D = x.shape
    C = prev_output.shape[1]
    nw = 32
    rows_per_w = B // nw
    k_drop = int(D * 0.2)
    mesh = plsc.VectorSubcoreMesh(core_axis_name="c", subcore_axis_name="s",
                                  num_cores=2, num_subcores=16)
    body = functools.partial(
        _body, rows_per_w=rows_per_w, nv=D // _L, cfull=C // _L, crem=C % _L,
        k_drop=k_drop)
    f = pl.kernel(
        body,
        out_type=jax.ShapeDtypeStruct((B, D), jnp.float32),
        mesh=mesh,
        compiler_params=pltpu.CompilerParams(needs_layout_passes=False),
        scratch_types=[
            pltpu.VMEM((C,), jnp.float32),
            pltpu.VMEM((D,), jnp.float32),
            pltpu.VMEM((2, D), jnp.float32),
            pltpu.VMEM((D,), jnp.float32),
            pltpu.VMEM((D,), jnp.float32),
            pltpu.VMEM((2,), jnp.int32),
            pltpu.VMEM((2048,), jnp.int32),
            pltpu.VMEM((1024,), jnp.int32),
            pltpu.SemaphoreType.DMA,
        ],
    )
    return f(x, prev_output, weight_matrix)


# speedup vs baseline: 13.3411x; 1.5632x over previous
import functools

import jax
import jax.numpy as jnp
from jax import lax
from jax.experimental import pallas as pl
from jax.experimental.pallas import tpu as pltpu
from jax.experimental.pallas import tpu_sc as plsc

_L = 16


def _body(x_hbm, prev_hbm, w_hbm, out_hbm,
          prev_v, x_v, w_v, s_v, o_v, idx_v, hist1_v, hist2_v, sem,
          *, rows_per_w, nv, cfull, crem, k_drop):
    lanes = jnp.arange(_L, dtype=jnp.int32)
    ones = jnp.ones((_L,), jnp.int32)
    ninf = jnp.full((_L,), -jnp.inf, jnp.float32)
    C = cfull * _L + crem
    BIGI = jnp.int32(1 << 20)

    wid = lax.axis_index("s") * 2 + lax.axis_index("c")
    base = wid * rows_per_w

    def scan_top(hist_ref, j_start, k_need):
        def cond(c):
            j, acc, found, p, kr = c
            return jnp.logical_not(found) & (j >= 0)

        def body(c):
            j, acc, found, p, kr = c
            h = hist_ref[pl.ds(j * _L, _L)]
            pre = plsc.cumsum(h)
            tot = jnp.max(pre)
            S = (acc + tot) - pre + h
            found_now = (acc + tot) >= kr
            mask = S >= kr
            cnt = jnp.max(plsc.all_reduce_population_count(mask))
            i = cnt - 1
            S_at = jnp.min(jnp.where(mask, S, jnp.int32(1 << 30)))
            h_at = jnp.max(jnp.where(lanes == i, h, 0))
            p_new = j * _L + i
            kr_new = kr - (S_at - h_at)
            return (j - 1, acc + tot, found_now,
                    jnp.where(found_now, p_new, p),
                    jnp.where(found_now, kr_new, kr))

        init = (j_start, jnp.int32(0), jnp.bool_(False),
                jnp.int32(0), k_need)
        _, _, _, p, kr = lax.while_loop(cond, body, init)
        return p, kr

    def clr1(j, _):
        hist1_v[pl.ds(j * _L, _L)] = jnp.zeros((_L,), jnp.int32)
        return 0

    def clr2(j, _):
        hist2_v[pl.ds(j * _L, _L)] = jnp.zeros((_L,), jnp.int32)
        return 0

    def row_step(i, _):
        r = base + i
        pltpu.sync_copy(prev_hbm.at[r], prev_v)

        def t2(j, c):
            a1, a2 = c
            v = prev_v[pl.ds(j * _L, _L)]
            a2 = jnp.maximum(a2, jnp.minimum(a1, v))
            a1 = jnp.maximum(a1, v)
            return a1, a2

        a1, a2 = lax.fori_loop(0, cfull, t2, (ninf, ninf), unroll=4)
        if crem:
            vt = prev_v[pl.ds(C - _L, _L)]
            vt = jnp.where(lanes >= (_L - crem), vt, ninf)
            a2 = jnp.maximum(a2, jnp.minimum(a1, vt))
            a1 = jnp.maximum(a1, vt)
        m1 = jnp.max(a1)
        f1 = jnp.max(plsc.all_reduce_ffs(a1 == m1))
        a1x = jnp.where(lanes == f1, ninf, a1)
        m2 = jnp.maximum(jnp.max(a1x), jnp.max(a2))

        nsweep = cfull + (1 if crem else 0)

        def idx_sweep(match_val, excl):
            def bodyf(j, c):
                off = jnp.where(j < cfull, j * _L, C - _L)
                lo = jnp.where(j < cfull, 0, _L - crem)
                v = prev_v[pl.ds(off, _L)]
                iv = lanes + off
                ok = (v == match_val) & (lanes >= lo) & (iv != excl)
                cand = jnp.where(ok, iv, BIGI)
                return jnp.minimum(c, jnp.min(cand))
            return lax.fori_loop(0, nsweep, bodyf, BIGI, unroll=2)

        i1 = idx_sweep(m1, jnp.int32(-1))
        i2 = idx_sweep(m2, i1)
        iv2 = jnp.where(lanes == 0, i1, i2)
        plsc.store_scatter(idx_v, [lanes], iv2, mask=lanes < 2)
        gather = pltpu.async_copy(w_hbm.at[idx_v], w_v, sem)
        pltpu.sync_copy(x_hbm.at[r], x_v)
        gather.wait()

        lax.fori_loop(0, hist1_v.shape[0] // _L, clr1, 0, unroll=8)
        lax.fori_loop(0, hist2_v.shape[0] // _L, clr2, 0, unroll=8)

        def sc_pass(j, bkmax):
            off = j * _L
            xv = x_v[pl.ds(off, _L)]
            dv = w_v[0, pl.ds(off, _L)] - w_v[1, pl.ds(off, _L)]
            s = jnp.abs(xv * dv)
            s_v[pl.ds(off, _L)] = s
            bits = plsc.bitcast(s, jnp.uint32)
            bk = (bits >> 20).astype(jnp.int32)
            plsc.addupdate_scatter(hist1_v, [bk], ones)
            return jnp.maximum(bkmax, bk)

        bkmax = lax.fori_loop(0, nv, sc_pass, jnp.zeros((_L,), jnp.int32),
                              unroll=4)
        p1, k1 = scan_top(hist1_v, jnp.max(bkmax) >> 4, jnp.int32(k_drop))
        p1u = p1.astype(jnp.uint32)

        def l2_pass(j, bkmax):
            s = s_v[pl.ds(j * _L, _L)]
            bits = plsc.bitcast(s, jnp.uint32)
            m = (bits >> 20) == p1u
            bk = ((bits >> 10) & jnp.uint32(1023)).astype(jnp.int32)
            plsc.addupdate_scatter(hist2_v, [bk], ones, mask=m)
            return jnp.maximum(bkmax, jnp.where(m, bk, 0))

        bkmax = lax.fori_loop(0, nv, l2_pass, jnp.zeros((_L,), jnp.int32),
                              unroll=4)
        p2, k2 = scan_top(hist2_v, jnp.max(bkmax) >> 4, k1)
        pfx2 = (p1u << jnp.uint32(10)) | p2.astype(jnp.uint32)

        lax.fori_loop(0, hist2_v.shape[0] // _L, clr2, 0, unroll=8)

        def l3_pass(j, bkmax):
            s = s_v[pl.ds(j * _L, _L)]
            bits = plsc.bitcast(s, jnp.uint32)
            m = (bits >> 10) == pfx2
            bk = (bits & jnp.uint32(1023)).astype(jnp.int32)
            plsc.addupdate_scatter(hist2_v, [bk], ones, mask=m)
            return jnp.maximum(bkmax, jnp.where(m, bk, 0))

        bkmax = lax.fori_loop(0, nv, l3_pass, jnp.zeros((_L,), jnp.int32),
                              unroll=4)
        p3, _ = scan_top(hist2_v, jnp.max(bkmax) >> 4, k2)
        thr = (pfx2 << jnp.uint32(10)) | p3.astype(jnp.uint32)

        def ap(j, _):
            off = j * _L
            s = s_v[pl.ds(off, _L)]
            bits = plsc.bitcast(s, jnp.uint32)
            xv = x_v[pl.ds(off, _L)]
            o_v[pl.ds(off, _L)] = jnp.where(bits < thr, xv, jnp.float32(0))
            return 0

        lax.fori_loop(0, nv, ap, 0, unroll=4)
        pltpu.sync_copy(o_v, out_hbm.at[r])
        return 0

    lax.fori_loop(0, rows_per_w, row_step, 0)


@jax.jit
def kernel(x, prev_output, weight_matrix):
    B, D = x.shape
    C = prev_output.shape[1]
    nw = 32
    rows_per_w = B // nw
    k_drop = int(D * 0.2)
    mesh = plsc.VectorSubcoreMesh(core_axis_name="c", subcore_axis_name="s",
                                  num_cores=2, num_subcores=16)
    body = functools.partial(
        _body, rows_per_w=rows_per_w, nv=D // _L, cfull=C // _L, crem=C % _L,
        k_drop=k_drop)
    f = pl.kernel(
        body,
        out_type=jax.ShapeDtypeStruct((B, D), jnp.float32),
        mesh=mesh,
        compiler_params=pltpu.CompilerParams(needs_layout_passes=False),
        scratch_types=[
            pltpu.VMEM((C,), jnp.float32),
            pltpu.VMEM((D,), jnp.float32),
            pltpu.VMEM((2, D), jnp.float32),
            pltpu.VMEM((D,), jnp.float32),
            pltpu.VMEM((D,), jnp.float32),
            pltpu.VMEM((2,), jnp.int32),
            pltpu.VMEM((2048,), jnp.int32),
            pltpu.VMEM((1024,), jnp.int32),
            pltpu.SemaphoreType.DMA,
        ],
    )
    return f(x, prev_output, weight_matrix)


# speedup vs baseline: 15.4059x; 1.1548x over previous
import functools

import jax
import jax.numpy as jnp
from jax import lax
from jax.experimental import pallas as pl
from jax.experimental.pallas import tpu as pltpu
from jax.experimental.pallas import tpu_sc as plsc

_L = 16


def _body(x_hbm, prev_hbm, w_hbm, out_hbm,
          prev0_v, prev1_v, x0_v, x1_v, w0_v, w1_v, s_v, o0_v, o1_v,
          idx0_v, idx1_v, hist1_v, hist2_v,
          semp, semx, semw, semo,
          *, rows_per_w, nv, cfull, crem, k_drop):
    prev_b = (prev0_v, prev1_v)
    x_b = (x0_v, x1_v)
    w_b = (w0_v, w1_v)
    o_b = (o0_v, o1_v)
    idx_b = (idx0_v, idx1_v)
    lanes = jnp.arange(_L, dtype=jnp.int32)
    ones = jnp.ones((_L,), jnp.int32)
    ninf = jnp.full((_L,), -jnp.inf, jnp.float32)
    C = cfull * _L + crem
    BIGI = jnp.int32(1 << 20)

    wid = lax.axis_index("s") * 2 + lax.axis_index("c")
    base = wid * rows_per_w

    def scan_top(hist_ref, j_start, k_need):
        def cond(c):
            j, acc, found, p, kr = c
            return jnp.logical_not(found) & (j >= 0)

        def body(c):
            j, acc, found, p, kr = c
            h = hist_ref[pl.ds(j * _L, _L)]
            pre = plsc.cumsum(h)
            tot = jnp.max(pre)
            S = (acc + tot) - pre + h
            found_now = (acc + tot) >= kr
            mask = S >= kr
            cnt = jnp.max(plsc.all_reduce_population_count(mask))
            i = cnt - 1
            S_at = jnp.min(jnp.where(mask, S, jnp.int32(1 << 30)))
            h_at = jnp.max(jnp.where(lanes == i, h, 0))
            p_new = j * _L + i
            kr_new = kr - (S_at - h_at)
            return (j - 1, acc + tot, found_now,
                    jnp.where(found_now, p_new, p),
                    jnp.where(found_now, kr_new, kr))

        init = (j_start, jnp.int32(0), jnp.bool_(False),
                jnp.int32(0), k_need)
        _, _, _, p, kr = lax.while_loop(cond, body, init)
        return p, kr

    def clr1(j, _):
        hist1_v[pl.ds(j * _L, _L)] = jnp.zeros((_L,), jnp.int32)
        return 0

    def clr2(j, _):
        hist2_v[pl.ds(j * _L, _L)] = jnp.zeros((_L,), jnp.int32)
        return 0

    nsweep = cfull + (1 if crem else 0)

    def top2_issue(r, b):
        prow = prev_b[b]

        def t2(j, c):
            a1, a2 = c
            v = prow[pl.ds(j * _L, _L)]
            a2 = jnp.maximum(a2, jnp.minimum(a1, v))
            a1 = jnp.maximum(a1, v)
            return a1, a2

        a1, a2 = lax.fori_loop(0, cfull, t2, (ninf, ninf), unroll=4)
        if crem:
            vt = prow[pl.ds(C - _L, _L)]
            vt = jnp.where(lanes >= (_L - crem), vt, ninf)
            a2 = jnp.maximum(a2, jnp.minimum(a1, vt))
            a1 = jnp.maximum(a1, vt)
        m1 = jnp.max(a1)
        f1 = jnp.max(plsc.all_reduce_ffs(a1 == m1))
        a1x = jnp.where(lanes == f1, ninf, a1)
        m2 = jnp.maximum(jnp.max(a1x), jnp.max(a2))

        def idx_sweep(match_val, excl):
            def bodyf(j, c):
                off = jnp.where(j < cfull, j * _L, C - _L)
                lo = jnp.where(j < cfull, 0, _L - crem)
                v = prow[pl.ds(off, _L)]
                iv = lanes + off
                ok = (v == match_val) & (lanes >= lo) & (iv != excl)
                cand = jnp.where(ok, iv, BIGI)
                return jnp.minimum(c, jnp.min(cand))
            return lax.fori_loop(0, nsweep, bodyf, BIGI, unroll=2)

        i1 = idx_sweep(m1, jnp.int32(-1))
        i2 = idx_sweep(m2, i1)
        iv2 = jnp.where(lanes == 0, i1, i2)
        plsc.store_scatter(idx_b[b], [lanes], iv2, mask=lanes < 2)
        pltpu.async_copy(w_hbm.at[idx_b[b]], w_b[b], semw)
        pltpu.async_copy(x_hbm.at[r], x_b[b], semx)

    def compute_row(r, b):
        xrow, wrow, orow = x_b[b], w_b[b], o_b[b]
        pltpu.make_async_copy(x_hbm.at[r], xrow, semx).wait()
        pltpu.make_async_copy(w_hbm.at[idx_b[b]], wrow, semw).wait()

        lax.fori_loop(0, hist1_v.shape[0] // _L, clr1, 0, unroll=8)
        lax.fori_loop(0, hist2_v.shape[0] // _L, clr2, 0, unroll=8)

        def sc_pass(j, bkmax):
            off = j * _L
            xv = xrow[pl.ds(off, _L)]
            dv = wrow[0, pl.ds(off, _L)] - wrow[1, pl.ds(off, _L)]
            s = jnp.abs(xv * dv)
            s_v[pl.ds(off, _L)] = s
            bits = plsc.bitcast(s, jnp.uint32)
            bk = (bits >> 20).astype(jnp.int32)
            plsc.addupdate_scatter(hist1_v, [bk], ones)
            return jnp.maximum(bkmax, bk)

        bkmax = lax.fori_loop(0, nv, sc_pass, jnp.zeros((_L,), jnp.int32),
                              unroll=4)
        p1, k1 = scan_top(hist1_v, jnp.max(bkmax) >> 4, jnp.int32(k_drop))
        p1u = p1.astype(jnp.uint32)

        def l2_pass(j, bkmax):
            s = s_v[pl.ds(j * _L, _L)]
            bits = plsc.bitcast(s, jnp.uint32)
            m = (bits >> 20) == p1u
            bk = ((bits >> 10) & jnp.uint32(1023)).astype(jnp.int32)
            plsc.addupdate_scatter(hist2_v, [bk], ones, mask=m)
            return jnp.maximum(bkmax, jnp.where(m, bk, 0))

        bkmax = lax.fori_loop(0, nv, l2_pass, jnp.zeros((_L,), jnp.int32),
                              unroll=4)
        p2, k2 = scan_top(hist2_v, jnp.max(bkmax) >> 4, k1)
        pfx2 = (p1u << jnp.uint32(10)) | p2.astype(jnp.uint32)

        lax.fori_loop(0, hist2_v.shape[0] // _L, clr2, 0, unroll=8)

        def l3_pass(j, bkmax):
            s = s_v[pl.ds(j * _L, _L)]
            bits = plsc.bitcast(s, jnp.uint32)
            m = (bits >> 10) == pfx2
            bk = (bits & jnp.uint32(1023)).astype(jnp.int32)
            plsc.addupdate_scatter(hist2_v, [bk], ones, mask=m)
            return jnp.maximum(bkmax, jnp.where(m, bk, 0))

        bkmax = lax.fori_loop(0, nv, l3_pass, jnp.zeros((_L,), jnp.int32),
                              unroll=4)
        p3, _ = scan_top(hist2_v, jnp.max(bkmax) >> 4, k2)
        thr = (pfx2 << jnp.uint32(10)) | p3.astype(jnp.uint32)

        @pl.when(r - base >= 2)
        def _():
            pltpu.make_async_copy(orow, out_hbm.at[r - 2], semo).wait()

        def ap(j, _):
            off = j * _L
            s = s_v[pl.ds(off, _L)]
            bits = plsc.bitcast(s, jnp.uint32)
            xv = xrow[pl.ds(off, _L)]
            orow[pl.ds(off, _L)] = jnp.where(bits < thr, xv, jnp.float32(0))
            return 0

        lax.fori_loop(0, nv, ap, 0, unroll=4)
        pltpu.async_copy(orow, out_hbm.at[r], semo)

    pltpu.sync_copy(prev_hbm.at[base], prev0_v)
    top2_issue(base, 0)
    pltpu.async_copy(prev_hbm.at[base + 1], prev1_v, semp)

    def outer(ro, _):
        for b in (0, 1):
            r = base + 2 * ro + b
            bn = 1 - b

            @pl.when(r - base < rows_per_w - 1)
            def _():
                pltpu.make_async_copy(prev_hbm.at[r + 1], prev_b[bn],
                                      semp).wait()
                top2_issue(r + 1, bn)

            @pl.when(r - base < rows_per_w - 2)
            def _():
                pltpu.async_copy(prev_hbm.at[r + 2], prev_b[b], semp)

            compute_row(r, b)
        return 0

    lax.fori_loop(0, rows_per_w // 2, outer, 0)

    pltpu.make_async_copy(o0_v, out_hbm.at[base + rows_per_w - 2],
                          semo).wait()
    pltpu.make_async_copy(o1_v, out_hbm.at[base + rows_per_w - 1],
                          semo).wait()


@jax.jit
def kernel(x, prev_output, weight_matrix):
    B, D = x.shape
    C = prev_output.shape[1]
    nw = 32
    rows_per_w = B // nw
    k_drop = int(D * 0.2)
    mesh = plsc.VectorSubcoreMesh(core_axis_name="c", subcore_axis_name="s")
    body = functools.partial(
        _body, rows_per_w=rows_per_w, nv=D // _L, cfull=C // _L, crem=C % _L,
        k_drop=k_drop)
    f = pl.kernel(
        body,
        out_type=jax.ShapeDtypeStruct((B, D), jnp.float32),
        mesh=mesh,
        compiler_params=pltpu.CompilerParams(needs_layout_passes=False),
        scratch_types=[
            pltpu.VMEM((C,), jnp.float32),
            pltpu.VMEM((C,), jnp.float32),
            pltpu.VMEM((D,), jnp.float32),
            pltpu.VMEM((D,), jnp.float32),
            pltpu.VMEM((2, D), jnp.float32),
            pltpu.VMEM((2, D), jnp.float32),
            pltpu.VMEM((D,), jnp.float32),
            pltpu.VMEM((D,), jnp.float32),
            pltpu.VMEM((D,), jnp.float32),
            pltpu.VMEM((2,), jnp.int32),
            pltpu.VMEM((2,), jnp.int32),
            pltpu.VMEM((2048,), jnp.int32),
            pltpu.VMEM((1024,), jnp.int32),
            pltpu.SemaphoreType.DMA,
            pltpu.SemaphoreType.DMA,
            pltpu.SemaphoreType.DMA,
            pltpu.SemaphoreType.DMA,
        ],
    )
    return f(x, prev_output, weight_matrix)


# speedup vs baseline: 17.3920x; 1.1289x over previous
import functools

import jax
import jax.numpy as jnp
from jax import lax
from jax.experimental import pallas as pl
from jax.experimental.pallas import tpu as pltpu
from jax.experimental.pallas import tpu_sc as plsc

_L = 16


def _body(x_hbm, prev_hbm, w_hbm, out_hbm,
          prev0_v, prev1_v, x0_v, x1_v, w0_v, w1_v, s_v, o0_v, o1_v,
          idx0_v, idx1_v, hist1_v, hist2_v,
          semp, semx, semw, semo,
          *, rows_per_w, nv, cfull, crem, k_drop):
    prev_b = (prev0_v, prev1_v)
    x_b = (x0_v, x1_v)
    w_b = (w0_v, w1_v)
    o_b = (o0_v, o1_v)
    idx_b = (idx0_v, idx1_v)
    lanes = jnp.arange(_L, dtype=jnp.int32)
    ones = jnp.ones((_L,), jnp.int32)
    c15 = jnp.full((_L,), _L - 1, jnp.int32)
    _dn = lax.GatherDimensionNumbers(
        offset_dims=(), collapsed_slice_dims=(0,), start_index_map=(0,))

    def take16(v, idx):
        return lax.gather(v, idx[:, None], _dn, (1,),
                          mode=lax.GatherScatterMode.PROMISE_IN_BOUNDS)
    ninf = jnp.full((_L,), -jnp.inf, jnp.float32)
    C = cfull * _L + crem
    BIGI = jnp.int32(1 << 20)

    wid = lax.axis_index("s") * 2 + lax.axis_index("c")
    base = wid * rows_per_w

    def scan_top(hist_ref, j_start, k_need):
        def cond(c):
            j, acc, found, p, kr = c
            return jnp.logical_not(found) & (j >= 0)

        def body(c):
            j, acc, found, p, kr = c
            h = hist_ref[pl.ds(j * _L, _L)]
            pre = plsc.cumsum(h)
            tot = take16(pre, c15)
            S = (acc + tot) - pre + h
            found_vec = (acc + tot) >= kr
            found_now = jnp.any(found_vec)
            mask = S >= kr
            i = jnp.maximum(plsc.all_reduce_population_count(mask) - 1, 0)
            S_at = take16(S, i)
            h_at = take16(h, i)
            p_new = j * _L + i
            kr_new = kr - (S_at - h_at)
            return (j - 1, acc + tot, found_now,
                    jnp.where(found_now, p_new, p),
                    jnp.where(found_now, kr_new, kr))

        init = (j_start, jnp.zeros((_L,), jnp.int32), jnp.bool_(False),
                jnp.zeros((_L,), jnp.int32), k_need)
        _, _, _, p, kr = lax.while_loop(cond, body, init)
        return p, kr

    def clr1(j, _):
        hist1_v[pl.ds(j * _L, _L)] = jnp.zeros((_L,), jnp.int32)
        return 0

    def clr2(j, _):
        hist2_v[pl.ds(j * _L, _L)] = jnp.zeros((_L,), jnp.int32)
        return 0

    nsweep = cfull + (1 if crem else 0)

    def top2_issue(r, b):
        prow = prev_b[b]

        def t2(j, c):
            a1, a2 = c
            v = prow[pl.ds(j * _L, _L)]
            a2 = jnp.maximum(a2, jnp.minimum(a1, v))
            a1 = jnp.maximum(a1, v)
            return a1, a2

        a1, a2 = lax.fori_loop(0, cfull, t2, (ninf, ninf), unroll=4)
        if crem:
            vt = prow[pl.ds(C - _L, _L)]
            vt = jnp.where(lanes >= (_L - crem), vt, ninf)
            a2 = jnp.maximum(a2, jnp.minimum(a1, vt))
            a1 = jnp.maximum(a1, vt)
        m1 = jnp.max(a1)
        f1 = jnp.max(plsc.all_reduce_ffs(a1 == m1))
        a1x = jnp.where(lanes == f1, ninf, a1)
        m2 = jnp.maximum(jnp.max(a1x), jnp.max(a2))

        BIGV = jnp.full((_L,), 1 << 20, jnp.int32)

        def bodyf(j, c):
            b1, b2, c1 = c
            off = jnp.where(j < cfull, j * _L, C - _L)
            lo = jnp.where(j < cfull, 0, _L - crem)
            v = prow[pl.ds(off, _L)]
            iv = lanes + off
            inb = lanes >= lo
            cand = jnp.where((v == m1) & inb, iv, BIGI)
            b2 = jnp.minimum(b2, jnp.maximum(b1, cand))
            b1 = jnp.minimum(b1, cand)
            c1 = jnp.minimum(c1, jnp.where((v == m2) & inb, iv, BIGI))
            return b1, b2, c1

        b1, b2, c1 = lax.fori_loop(0, nsweep, bodyf, (BIGV, BIGV, BIGV),
                                   unroll=2)
        i1 = jnp.min(b1)
        fb = jnp.max(plsc.all_reduce_ffs(b1 == i1))
        b1x = jnp.where(lanes == fb, BIGI, b1)
        i1b = jnp.minimum(jnp.min(b1x), jnp.min(b2))
        i2 = jnp.where(m2 == m1, i1b, jnp.min(c1))
        iv2 = jnp.where(lanes == 0, i1, i2)
        plsc.store_scatter(idx_b[b], [lanes], iv2, mask=lanes < 2)
        pltpu.async_copy(w_hbm.at[idx_b[b]], w_b[b], semw)
        pltpu.async_copy(x_hbm.at[r], x_b[b], semx)

    def compute_row(r, b):
        xrow, wrow, orow = x_b[b], w_b[b], o_b[b]
        pltpu.make_async_copy(x_hbm.at[r], xrow, semx).wait()
        pltpu.make_async_copy(w_hbm.at[idx_b[b]], wrow, semw).wait()

        lax.fori_loop(0, hist1_v.shape[0] // _L, clr1, 0, unroll=8)
        lax.fori_loop(0, hist2_v.shape[0] // _L, clr2, 0, unroll=8)

        def sc_pass(j, bkmax):
            off = j * _L
            xv = xrow[pl.ds(off, _L)]
            dv = wrow[0, pl.ds(off, _L)] - wrow[1, pl.ds(off, _L)]
            s = jnp.abs(xv * dv)
            s_v[pl.ds(off, _L)] = s
            bits = plsc.bitcast(s, jnp.uint32)
            bk = (bits >> 20).astype(jnp.int32)
            plsc.addupdate_scatter(hist1_v, [bk], ones)
            return jnp.maximum(bkmax, bk)

        bkmax = lax.fori_loop(0, nv, sc_pass, jnp.zeros((_L,), jnp.int32),
                              unroll=4)
        p1, k1 = scan_top(hist1_v, jnp.max(bkmax) >> 4,
                          jnp.full((_L,), k_drop, jnp.int32))
        p1u = p1.astype(jnp.uint32)

        def l2_pass(j, bkmax):
            s = s_v[pl.ds(j * _L, _L)]
            bits = plsc.bitcast(s, jnp.uint32)
            m = (bits >> 20) == p1u
            bk = ((bits >> 10) & jnp.uint32(1023)).astype(jnp.int32)
            plsc.addupdate_scatter(hist2_v, [bk], ones, mask=m)
            return jnp.maximum(bkmax, jnp.where(m, bk, 0))

        bkmax = lax.fori_loop(0, nv, l2_pass, jnp.zeros((_L,), jnp.int32),
                              unroll=4)
        p2, k2 = scan_top(hist2_v, jnp.max(bkmax) >> 4, k1)
        pfx2 = (p1u << jnp.uint32(10)) | p2.astype(jnp.uint32)

        lax.fori_loop(0, hist2_v.shape[0] // _L, clr2, 0, unroll=8)

        def l3_pass(j, bkmax):
            s = s_v[pl.ds(j * _L, _L)]
            bits = plsc.bitcast(s, jnp.uint32)
            m = (bits >> 10) == pfx2
            bk = (bits & jnp.uint32(1023)).astype(jnp.int32)
            plsc.addupdate_scatter(hist2_v, [bk], ones, mask=m)
            return jnp.maximum(bkmax, jnp.where(m, bk, 0))

        bkmax = lax.fori_loop(0, nv, l3_pass, jnp.zeros((_L,), jnp.int32),
                              unroll=4)
        p3, _ = scan_top(hist2_v, jnp.max(bkmax) >> 4, k2)
        thr = (pfx2 << jnp.uint32(10)) | p3.astype(jnp.uint32)

        @pl.when(r - base >= 2)
        def _():
            pltpu.make_async_copy(orow, out_hbm.at[r - 2], semo).wait()

        def ap(j, _):
            off = j * _L
            s = s_v[pl.ds(off, _L)]
            bits = plsc.bitcast(s, jnp.uint32)
            xv = xrow[pl.ds(off, _L)]
            orow[pl.ds(off, _L)] = jnp.where(bits < thr, xv, jnp.float32(0))
            return 0

        lax.fori_loop(0, nv, ap, 0, unroll=4)
        pltpu.async_copy(orow, out_hbm.at[r], semo)

    pltpu.sync_copy(prev_hbm.at[base], prev0_v)
    top2_issue(base, 0)
    pltpu.async_copy(prev_hbm.at[base + 1], prev1_v, semp)

    def outer(ro, _):
        for b in (0, 1):
            r = base + 2 * ro + b
            bn = 1 - b

            @pl.when(r - base < rows_per_w - 1)
            def _():
                pltpu.make_async_copy(prev_hbm.at[r + 1], prev_b[bn],
                                      semp).wait()
                top2_issue(r + 1, bn)

            @pl.when(r - base < rows_per_w - 2)
            def _():
                pltpu.async_copy(prev_hbm.at[r + 2], prev_b[b], semp)

            compute_row(r, b)
        return 0

    lax.fori_loop(0, rows_per_w // 2, outer, 0)

    pltpu.make_async_copy(o0_v, out_hbm.at[base + rows_per_w - 2],
                          semo).wait()
    pltpu.make_async_copy(o1_v, out_hbm.at[base + rows_per_w - 1],
                          semo).wait()


@jax.jit
def kernel(x, prev_output, weight_matrix):
    B, D = x.shape
    C = prev_output.shape[1]
    nw = 32
    rows_per_w = B // nw
    k_drop = int(D * 0.2)
    mesh = plsc.VectorSubcoreMesh(core_axis_name="c", subcore_axis_name="s")
    body = functools.partial(
        _body, rows_per_w=rows_per_w, nv=D // _L, cfull=C // _L, crem=C % _L,
        k_drop=k_drop)
    f = pl.kernel(
        body,
        out_type=jax.ShapeDtypeStruct((B, D), jnp.float32),
        mesh=mesh,
        compiler_params=pltpu.CompilerParams(needs_layout_passes=False),
        scratch_types=[
            pltpu.VMEM((C,), jnp.float32),
            pltpu.VMEM((C,), jnp.float32),
            pltpu.VMEM((D,), jnp.float32),
            pltpu.VMEM((D,), jnp.float32),
            pltpu.VMEM((2, D), jnp.float32),
            pltpu.VMEM((2, D), jnp.float32),
            pltpu.VMEM((D,), jnp.float32),
            pltpu.VMEM((D,), jnp.float32),
            pltpu.VMEM((D,), jnp.float32),
            pltpu.VMEM((2,), jnp.int32),
            pltpu.VMEM((2,), jnp.int32),
            pltpu.VMEM((2048,), jnp.int32),
            pltpu.VMEM((1024,), jnp.int32),
            pltpu.SemaphoreType.DMA,
            pltpu.SemaphoreType.DMA,
            pltpu.SemaphoreType.DMA,
            pltpu.SemaphoreType.DMA,
        ],
    )
    return f(x, prev_output, weight_matrix)


# speedup vs baseline: 46.5560x; 2.6769x over previous
import functools

import jax
import jax.numpy as jnp
from jax import lax
from jax.experimental import pallas as pl
from jax.experimental.pallas import tpu as pltpu
from jax.experimental.pallas import tpu_sc as plsc

_L = 16


def _body(x_hbm, prev_hbm, w_hbm, out_hbm,
          prev0_v, prev1_v, x0_v, x1_v, w0_v, w1_v, s_v, o0_v, o1_v,
          idx0_v, idx1_v, hist1_v, hist2_v,
          semp, semx, semw, semo,
          *, rows_per_w, nv, cfull, crem, k_drop):
    prev_b = (prev0_v, prev1_v)
    x_b = (x0_v, x1_v)
    w_b = (w0_v, w1_v)
    o_b = (o0_v, o1_v)
    idx_b = (idx0_v, idx1_v)
    lanes = jnp.arange(_L, dtype=jnp.int32)
    ones = jnp.ones((_L,), jnp.int32)
    c15 = jnp.full((_L,), _L - 1, jnp.int32)
    _dn = lax.GatherDimensionNumbers(
        offset_dims=(), collapsed_slice_dims=(0,), start_index_map=(0,))

    def take16(v, idx):
        return lax.gather(v, idx[:, None], _dn, (1,),
                          mode=lax.GatherScatterMode.PROMISE_IN_BOUNDS)
    ninf = jnp.full((_L,), -jnp.inf, jnp.float32)
    C = cfull * _L + crem
    BIGI = jnp.int32(1 << 20)

    wid = lax.axis_index("s") * 2 + lax.axis_index("c")
    base = wid * rows_per_w

    def scan_top(hist_ref, j_start, k_need):
        def cond(c):
            j, acc, found, p, kr = c
            return jnp.logical_not(found) & (j >= 0)

        def body(c):
            j, acc, found, p, kr = c
            h = hist_ref[pl.ds(j * _L, _L)]
            pre = plsc.cumsum(h)
            tot = take16(pre, c15)
            S = (acc + tot) - pre + h
            found_vec = (acc + tot) >= kr
            found_now = jnp.any(found_vec)
            mask = S >= kr
            i = jnp.maximum(plsc.all_reduce_population_count(mask) - 1, 0)
            S_at = take16(S, i)
            h_at = take16(h, i)
            p_new = j * _L + i
            kr_new = kr - (S_at - h_at)
            return (j - 1, acc + tot, found_now,
                    jnp.where(found_now, p_new, p),
                    jnp.where(found_now, kr_new, kr))

        init = (j_start, jnp.zeros((_L,), jnp.int32), jnp.bool_(False),
                jnp.zeros((_L,), jnp.int32), k_need)
        _, _, _, p, kr = lax.while_loop(cond, body, init)
        return p, kr

    def clr1(j):
        hist1_v[pl.ds(j * _L, _L)] = jnp.zeros((_L,), jnp.int32)

    def clr2(j):
        hist2_v[pl.ds(j * _L, _L)] = jnp.zeros((_L,), jnp.int32)

    nsweep = cfull + (1 if crem else 0)

    def top2_issue(r, b):
        prow = prev_b[b]

        def t2(j, c):
            a1, a2 = c
            v = prow[pl.ds(j * _L, _L)]
            a2 = jnp.maximum(a2, jnp.minimum(a1, v))
            a1 = jnp.maximum(a1, v)
            return a1, a2

        a1, a2 = plsc.parallel_loop(0, cfull, carry=(ninf, ninf), unroll=4)(t2)
        if crem:
            vt = prow[pl.ds(C - _L, _L)]
            vt = jnp.where(lanes >= (_L - crem), vt, ninf)
            a2 = jnp.maximum(a2, jnp.minimum(a1, vt))
            a1 = jnp.maximum(a1, vt)
        m1 = jnp.max(a1)
        f1 = jnp.max(plsc.all_reduce_ffs(a1 == m1))
        a1x = jnp.where(lanes == f1, ninf, a1)
        m2 = jnp.maximum(jnp.max(a1x), jnp.max(a2))

        BIGV = jnp.full((_L,), 1 << 20, jnp.int32)

        def bodyf(j, c):
            b1, b2, c1 = c
            off = jnp.where(j < cfull, j * _L, C - _L)
            lo = jnp.where(j < cfull, 0, _L - crem)
            v = prow[pl.ds(off, _L)]
            iv = lanes + off
            inb = lanes >= lo
            cand = jnp.where((v == m1) & inb, iv, BIGI)
            b2 = jnp.minimum(b2, jnp.maximum(b1, cand))
            b1 = jnp.minimum(b1, cand)
            c1 = jnp.minimum(c1, jnp.where((v == m2) & inb, iv, BIGI))
            return b1, b2, c1

        b1, b2, c1 = plsc.parallel_loop(
            0, nsweep, carry=(BIGV, BIGV, BIGV), unroll=2)(bodyf)
        i1 = jnp.min(b1)
        fb = jnp.max(plsc.all_reduce_ffs(b1 == i1))
        b1x = jnp.where(lanes == fb, BIGI, b1)
        i1b = jnp.minimum(jnp.min(b1x), jnp.min(b2))
        i2 = jnp.where(m2 == m1, i1b, jnp.min(c1))
        iv2 = jnp.where(lanes == 0, i1, i2)
        plsc.store_scatter(idx_b[b], [lanes], iv2, mask=lanes < 2)
        pltpu.async_copy(w_hbm.at[idx_b[b]], w_b[b], semw)
        pltpu.async_copy(x_hbm.at[r], x_b[b], semx)

    def compute_row(r, b):
        xrow, wrow, orow = x_b[b], w_b[b], o_b[b]
        pltpu.make_async_copy(x_hbm.at[r], xrow, semx).wait()
        pltpu.make_async_copy(w_hbm.at[idx_b[b]], wrow, semw).wait()

        plsc.parallel_loop(0, hist1_v.shape[0] // _L, unroll=8)(clr1)
        plsc.parallel_loop(0, hist2_v.shape[0] // _L, unroll=8)(clr2)

        def sc_pass(j, bkmax):
            off = j * _L
            xv = xrow[pl.ds(off, _L)]
            dv = wrow[0, pl.ds(off, _L)] - wrow[1, pl.ds(off, _L)]
            s = jnp.abs(xv * dv)
            s_v[pl.ds(off, _L)] = s
            bits = plsc.bitcast(s, jnp.uint32)
            bk = (bits >> 20).astype(jnp.int32)
            plsc.addupdate_scatter(hist1_v, [bk], ones)
            return jnp.maximum(bkmax, bk)

        bkmax = plsc.parallel_loop(
            0, nv, carry=jnp.zeros((_L,), jnp.int32), unroll=4)(sc_pass)
        p1, k1 = scan_top(hist1_v, jnp.max(bkmax) >> 4,
                          jnp.full((_L,), k_drop, jnp.int32))
        p1u = p1.astype(jnp.uint32)

        def l2_pass(j, bkmax):
            s = s_v[pl.ds(j * _L, _L)]
            bits = plsc.bitcast(s, jnp.uint32)
            m = (bits >> 20) == p1u
            bk = ((bits >> 10) & jnp.uint32(1023)).astype(jnp.int32)
            plsc.addupdate_scatter(hist2_v, [bk], ones, mask=m)
            return jnp.maximum(bkmax, jnp.where(m, bk, 0))

        bkmax = plsc.parallel_loop(
            0, nv, carry=jnp.zeros((_L,), jnp.int32), unroll=4)(l2_pass)
        p2, k2 = scan_top(hist2_v, jnp.max(bkmax) >> 4, k1)
        pfx2 = (p1u << jnp.uint32(10)) | p2.astype(jnp.uint32)

        plsc.parallel_loop(0, hist2_v.shape[0] // _L, unroll=8)(clr2)

        def l3_pass(j, bkmax):
            s = s_v[pl.ds(j * _L, _L)]
            bits = plsc.bitcast(s, jnp.uint32)
            m = (bits >> 10) == pfx2
            bk = (bits & jnp.uint32(1023)).astype(jnp.int32)
            plsc.addupdate_scatter(hist2_v, [bk], ones, mask=m)
            return jnp.maximum(bkmax, jnp.where(m, bk, 0))

        bkmax = plsc.parallel_loop(
            0, nv, carry=jnp.zeros((_L,), jnp.int32), unroll=4)(l3_pass)
        p3, _ = scan_top(hist2_v, jnp.max(bkmax) >> 4, k2)
        thr = (pfx2 << jnp.uint32(10)) | p3.astype(jnp.uint32)

        @pl.when(r - base >= 2)
        def _():
            pltpu.make_async_copy(orow, out_hbm.at[r - 2], semo).wait()

        def ap(j):
            off = j * _L
            s = s_v[pl.ds(off, _L)]
            bits = plsc.bitcast(s, jnp.uint32)
            xv = xrow[pl.ds(off, _L)]
            orow[pl.ds(off, _L)] = jnp.where(bits < thr, xv, jnp.float32(0))

        plsc.parallel_loop(0, nv, unroll=4)(ap)
        pltpu.async_copy(orow, out_hbm.at[r], semo)

    pltpu.sync_copy(prev_hbm.at[base], prev0_v)
    top2_issue(base, 0)
    pltpu.async_copy(prev_hbm.at[base + 1], prev1_v, semp)

    def outer(ro, _):
        for b in (0, 1):
            r = base + 2 * ro + b
            bn = 1 - b

            @pl.when(r - base < rows_per_w - 1)
            def _():
                pltpu.make_async_copy(prev_hbm.at[r + 1], prev_b[bn],
                                      semp).wait()
                top2_issue(r + 1, bn)

            @pl.when(r - base < rows_per_w - 2)
            def _():
                pltpu.async_copy(prev_hbm.at[r + 2], prev_b[b], semp)

            compute_row(r, b)
        return 0

    lax.fori_loop(0, rows_per_w // 2, outer, 0)

    pltpu.make_async_copy(o0_v, out_hbm.at[base + rows_per_w - 2],
                          semo).wait()
    pltpu.make_async_copy(o1_v, out_hbm.at[base + rows_per_w - 1],
                          semo).wait()


@jax.jit
def kernel(x, prev_output, weight_matrix):
    B, D = x.shape
    C = prev_output.shape[1]
    nw = 32
    rows_per_w = B // nw
    k_drop = int(D * 0.2)
    mesh = plsc.VectorSubcoreMesh(core_axis_name="c", subcore_axis_name="s")
    body = functools.partial(
        _body, rows_per_w=rows_per_w, nv=D // _L, cfull=C // _L, crem=C % _L,
        k_drop=k_drop)
    f = pl.kernel(
        body,
        out_type=jax.ShapeDtypeStruct((B, D), jnp.float32),
        mesh=mesh,
        compiler_params=pltpu.CompilerParams(needs_layout_passes=False),
        scratch_types=[
            pltpu.VMEM((C,), jnp.float32),
            pltpu.VMEM((C,), jnp.float32),
            pltpu.VMEM((D,), jnp.float32),
            pltpu.VMEM((D,), jnp.float32),
            pltpu.VMEM((2, D), jnp.float32),
            pltpu.VMEM((2, D), jnp.float32),
            pltpu.VMEM((D,), jnp.float32),
            pltpu.VMEM((D,), jnp.float32),
            pltpu.VMEM((D,), jnp.float32),
            pltpu.VMEM((2,), jnp.int32),
            pltpu.VMEM((2,), jnp.int32),
            pltpu.VMEM((2048,), jnp.int32),
            pltpu.VMEM((1024,), jnp.int32),
            pltpu.SemaphoreType.DMA,
            pltpu.SemaphoreType.DMA,
            pltpu.SemaphoreType.DMA,
            pltpu.SemaphoreType.DMA,
        ],
    )
    return f(x, prev_output, weight_matrix)


# speedup vs baseline: 47.9096x; 1.0291x over previous
import functools

import jax
import jax.numpy as jnp
from jax import lax
from jax.experimental import pallas as pl
from jax.experimental.pallas import tpu as pltpu
from jax.experimental.pallas import tpu_sc as plsc

_L = 16


def _body(x_hbm, prev_hbm, w_hbm, out_hbm,
          prev0_v, prev1_v, x0_v, x1_v, w0_v, w1_v, s_v, o0_v, o1_v,
          idx0_v, idx1_v, hist1_v, hist2_v,
          semp, semx, semw, semo,
          *, rows_per_w, nv, cfull, crem, k_drop):
    prev_b = (prev0_v, prev1_v)
    x_b = (x0_v, x1_v)
    w_b = (w0_v, w1_v)
    o_b = (o0_v, o1_v)
    idx_b = (idx0_v, idx1_v)
    lanes = jnp.arange(_L, dtype=jnp.int32)
    ones = jnp.ones((_L,), jnp.int32)
    c15 = jnp.full((_L,), _L - 1, jnp.int32)
    _dn = lax.GatherDimensionNumbers(
        offset_dims=(), collapsed_slice_dims=(0,), start_index_map=(0,))

    def take16(v, idx):
        return lax.gather(v, idx[:, None], _dn, (1,),
                          mode=lax.GatherScatterMode.PROMISE_IN_BOUNDS)
    ninf = jnp.full((_L,), -jnp.inf, jnp.float32)
    C = cfull * _L + crem
    BIGI = jnp.int32(1 << 20)

    wid = lax.axis_index("s") * 2 + lax.axis_index("c")
    base = wid * rows_per_w

    def scan_top(hist_ref, j_start, k_need):
        def cond(c):
            j, acc, found, p, kr = c
            return jnp.logical_not(found) & (j >= 0)

        def body(c):
            j, acc, found, p, kr = c
            h = hist_ref[pl.ds(j * _L, _L)]
            pre = plsc.cumsum(h)
            tot = take16(pre, c15)
            S = (acc + tot) - pre + h
            found_vec = (acc + tot) >= kr
            found_now = jnp.any(found_vec)
            mask = S >= kr
            i = jnp.maximum(plsc.all_reduce_population_count(mask) - 1, 0)
            S_at = take16(S, i)
            h_at = take16(h, i)
            p_new = j * _L + i
            kr_new = kr - (S_at - h_at)
            return (j - 1, acc + tot, found_now,
                    jnp.where(found_now, p_new, p),
                    jnp.where(found_now, kr_new, kr))

        init = (j_start, jnp.zeros((_L,), jnp.int32), jnp.bool_(False),
                jnp.zeros((_L,), jnp.int32), k_need)
        _, _, _, p, kr = lax.while_loop(cond, body, init)
        return p, kr

    def clr1(j):
        hist1_v[pl.ds(j * _L, _L)] = jnp.zeros((_L,), jnp.int32)

    def clr2(j):
        hist2_v[pl.ds(j * _L, _L)] = jnp.zeros((_L,), jnp.int32)

    nsweep = cfull + (1 if crem else 0)

    def top2_issue(r, b):
        prow = prev_b[b]

        def t2(j, c):
            a1, a2 = c
            v = prow[pl.ds(j * _L, _L)]
            a2 = jnp.maximum(a2, jnp.minimum(a1, v))
            a1 = jnp.maximum(a1, v)
            return a1, a2

        a1, a2 = plsc.parallel_loop(0, cfull, carry=(ninf, ninf), unroll=8)(t2)
        if crem:
            vt = prow[pl.ds(C - _L, _L)]
            vt = jnp.where(lanes >= (_L - crem), vt, ninf)
            a2 = jnp.maximum(a2, jnp.minimum(a1, vt))
            a1 = jnp.maximum(a1, vt)
        m1 = jnp.max(a1)
        f1 = jnp.max(plsc.all_reduce_ffs(a1 == m1))
        a1x = jnp.where(lanes == f1, ninf, a1)
        m2 = jnp.maximum(jnp.max(a1x), jnp.max(a2))

        BIGV = jnp.full((_L,), 1 << 20, jnp.int32)

        def bodyf(j, c):
            b1, b2, c1 = c
            off = jnp.where(j < cfull, j * _L, C - _L)
            lo = jnp.where(j < cfull, 0, _L - crem)
            v = prow[pl.ds(off, _L)]
            iv = lanes + off
            inb = lanes >= lo
            cand = jnp.where((v == m1) & inb, iv, BIGI)
            b2 = jnp.minimum(b2, jnp.maximum(b1, cand))
            b1 = jnp.minimum(b1, cand)
            c1 = jnp.minimum(c1, jnp.where((v == m2) & inb, iv, BIGI))
            return b1, b2, c1

        b1, b2, c1 = plsc.parallel_loop(
            0, nsweep, carry=(BIGV, BIGV, BIGV), unroll=4)(bodyf)
        i1 = jnp.min(b1)
        fb = jnp.max(plsc.all_reduce_ffs(b1 == i1))
        b1x = jnp.where(lanes == fb, BIGI, b1)
        i1b = jnp.minimum(jnp.min(b1x), jnp.min(b2))
        i2 = jnp.where(m2 == m1, i1b, jnp.min(c1))
        iv2 = jnp.where(lanes == 0, i1, i2)
        plsc.store_scatter(idx_b[b], [lanes], iv2, mask=lanes < 2)
        pltpu.async_copy(w_hbm.at[idx_b[b]], w_b[b], semw)
        pltpu.async_copy(x_hbm.at[r], x_b[b], semx)

    def compute_row(r, b):
        xrow, wrow, orow = x_b[b], w_b[b], o_b[b]
        pltpu.make_async_copy(x_hbm.at[r], xrow, semx).wait()
        pltpu.make_async_copy(w_hbm.at[idx_b[b]], wrow, semw).wait()

        plsc.parallel_loop(0, hist1_v.shape[0] // _L, unroll=8)(clr1)
        plsc.parallel_loop(0, hist2_v.shape[0] // _L, unroll=8)(clr2)

        def sc_pass(j, bkmax):
            off = j * _L
            xv = xrow[pl.ds(off, _L)]
            dv = wrow[0, pl.ds(off, _L)] - wrow[1, pl.ds(off, _L)]
            s = jnp.abs(xv * dv)
            s_v[pl.ds(off, _L)] = s
            bits = plsc.bitcast(s, jnp.uint32)
            bk = (bits >> 20).astype(jnp.int32)
            plsc.addupdate_scatter(hist1_v, [bk], ones)
            return jnp.maximum(bkmax, bk)

        bkmax = plsc.parallel_loop(
            0, nv, carry=jnp.zeros((_L,), jnp.int32), unroll=8)(sc_pass)
        p1, k1 = scan_top(hist1_v, jnp.max(bkmax) >> 4,
                          jnp.full((_L,), k_drop, jnp.int32))
        p1u = p1.astype(jnp.uint32)

        def l2_pass(j, bkmax):
            s = s_v[pl.ds(j * _L, _L)]
            bits = plsc.bitcast(s, jnp.uint32)
            m = (bits >> 20) == p1u
            bk = ((bits >> 10) & jnp.uint32(1023)).astype(jnp.int32)
            plsc.addupdate_scatter(hist2_v, [bk], ones, mask=m)
            return jnp.maximum(bkmax, jnp.where(m, bk, 0))

        bkmax = plsc.parallel_loop(
            0, nv, carry=jnp.zeros((_L,), jnp.int32), unroll=8)(l2_pass)
        p2, k2 = scan_top(hist2_v, jnp.max(bkmax) >> 4, k1)
        pfx2 = (p1u << jnp.uint32(10)) | p2.astype(jnp.uint32)

        plsc.parallel_loop(0, hist2_v.shape[0] // _L, unroll=8)(clr2)

        def l3_pass(j, bkmax):
            s = s_v[pl.ds(j * _L, _L)]
            bits = plsc.bitcast(s, jnp.uint32)
            m = (bits >> 10) == pfx2
            bk = (bits & jnp.uint32(1023)).astype(jnp.int32)
            plsc.addupdate_scatter(hist2_v, [bk], ones, mask=m)
            return jnp.maximum(bkmax, jnp.where(m, bk, 0))

        bkmax = plsc.parallel_loop(
            0, nv, carry=jnp.zeros((_L,), jnp.int32), unroll=8)(l3_pass)
        p3, _ = scan_top(hist2_v, jnp.max(bkmax) >> 4, k2)
        thr = (pfx2 << jnp.uint32(10)) | p3.astype(jnp.uint32)

        @pl.when(r - base >= 2)
        def _():
            pltpu.make_async_copy(orow, out_hbm.at[r - 2], semo).wait()

        def ap(j):
            off = j * _L
            s = s_v[pl.ds(off, _L)]
            bits = plsc.bitcast(s, jnp.uint32)
            xv = xrow[pl.ds(off, _L)]
            orow[pl.ds(off, _L)] = jnp.where(bits < thr, xv, jnp.float32(0))

        plsc.parallel_loop(0, nv, unroll=8)(ap)
        pltpu.async_copy(orow, out_hbm.at[r], semo)

    pltpu.sync_copy(prev_hbm.at[base], prev0_v)
    top2_issue(base, 0)
    pltpu.async_copy(prev_hbm.at[base + 1], prev1_v, semp)

    def outer(ro, _):
        for b in (0, 1):
            r = base + 2 * ro + b
            bn = 1 - b

            @pl.when(r - base < rows_per_w - 1)
            def _():
                pltpu.make_async_copy(prev_hbm.at[r + 1], prev_b[bn],
                                      semp).wait()
                top2_issue(r + 1, bn)

            @pl.when(r - base < rows_per_w - 2)
            def _():
                pltpu.async_copy(prev_hbm.at[r + 2], prev_b[b], semp)

            compute_row(r, b)
        return 0

    lax.fori_loop(0, rows_per_w // 2, outer, 0)

    pltpu.make_async_copy(o0_v, out_hbm.at[base + rows_per_w - 2],
                          semo).wait()
    pltpu.make_async_copy(o1_v, out_hbm.at[base + rows_per_w - 1],
                          semo).wait()


@jax.jit
def kernel(x, prev_output, weight_matrix):
    B, D = x.shape
    C = prev_output.shape[1]
    nw = 32
    rows_per_w = B // nw
    k_drop = int(D * 0.2)
    mesh = plsc.VectorSubcoreMesh(core_axis_name="c", subcore_axis_name="s")
    body = functools.partial(
        _body, rows_per_w=rows_per_w, nv=D // _L, cfull=C // _L, crem=C % _L,
        k_drop=k_drop)
    f = pl.kernel(
        body,
        out_type=jax.ShapeDtypeStruct((B, D), jnp.float32),
        mesh=mesh,
        compiler_params=pltpu.CompilerParams(needs_layout_passes=False),
        scratch_types=[
            pltpu.VMEM((C,), jnp.float32),
            pltpu.VMEM((C,), jnp.float32),
            pltpu.VMEM((D,), jnp.float32),
            pltpu.VMEM((D,), jnp.float32),
            pltpu.VMEM((2, D), jnp.float32),
            pltpu.VMEM((2, D), jnp.float32),
            pltpu.VMEM((D,), jnp.float32),
            pltpu.VMEM((D,), jnp.float32),
            pltpu.VMEM((D,), jnp.float32),
            pltpu.VMEM((2,), jnp.int32),
            pltpu.VMEM((2,), jnp.int32),
            pltpu.VMEM((2048,), jnp.int32),
            pltpu.VMEM((1024,), jnp.int32),
            pltpu.SemaphoreType.DMA,
            pltpu.SemaphoreType.DMA,
            pltpu.SemaphoreType.DMA,
            pltpu.SemaphoreType.DMA,
        ],
    )
    return f(x, prev_output, weight_matrix)
